# Initial kernel scaffold; baseline (speedup 1.0000x reference)
import jax
import jax.numpy as jnp
from jax.experimental import pallas as pl


def kernel(x, edge_index, batch, W1, b1, W2, b2, W3, b3, fc_w):
    raise NotImplementedError("write your pallas kernel here")


# speedup vs baseline: 11.4683x; 11.4683x over previous
import functools

import jax
import jax.numpy as jnp
from jax import lax
from jax.experimental import pallas as pl
from jax.experimental.pallas import tpu as pltpu
from jax.experimental.pallas import tpu_sc as plsc

N = 50000
E = 800000
H = 64
G = 64

EPAD = 819200
EROWS = EPAD // 128
NPAD = N + 8
NB = 1000
NGRID = N // NB
TPN = N // 16
ZROWS = TPN // 5


def _fill_zero_rows(ref, nrows, width):
    def body(j, carry):
        for w0 in range(0, width, 16):
            ref[j, pl.ds(w0, 16)] = jnp.zeros((16,), jnp.float32)
        return carry
    lax.fori_loop(0, nrows, body, 0)


def _zero_acc_slice(acc, zbuf, sid):
    base = sid * TPN
    for k in range(5):
        pltpu.sync_copy(zbuf, acc.at[pl.ds(base + k * ZROWS, ZROWS)])


def _sc_deg(dst2):
    mesh = plsc.VectorSubcoreMesh(core_axis_name="c", subcore_axis_name="s")

    @functools.partial(
        pl.kernel,
        out_type=jax.ShapeDtypeStruct((2, 16, TPN, 16), jnp.float32),
        mesh=mesh,
        compiler_params=pltpu.CompilerParams(use_tc_tiling_on_sc=False),
        scratch_types=[
            pltpu.VMEM_SHARED((NPAD, 16), jnp.float32),
            pltpu.VMEM((8, 128), jnp.int32),
            pltpu.VMEM((128, 16), jnp.float32),
            pltpu.VMEM((ZROWS, 16), jnp.float32),
        ],
    )
    def deg_kernel(dst_hbm, cnt_hbm, acc, dstv, onesv, zbuf):
        c = lax.axis_index("c")
        sid = lax.axis_index("s")

        _fill_zero_rows(zbuf, ZROWS, 16)

        def fill_ones(j, carry):
            onesv[j, pl.ds(0, 16)] = jnp.ones((16,), jnp.float32)
            return carry
        lax.fori_loop(0, 128, fill_ones, 0)

        _zero_acc_slice(acc, zbuf, sid)
        plsc.subcore_barrier()

        def body(j, carry):
            row0 = c * (EROWS // 2) + sid * 200 + j * 8
            pltpu.sync_copy(dst_hbm.at[pl.ds(row0, 8)], dstv)
            for r in range(8):
                pltpu.sync_copy(onesv, acc.at[dstv.at[r]], add=True)
            return carry
        lax.fori_loop(0, 25, body, 0)

        plsc.subcore_barrier()
        pltpu.sync_copy(acc.at[pl.ds(sid * TPN, TPN)],
                        cnt_hbm.at[c, sid])

    return deg_kernel(dst2).reshape(2, N, 16)


def _sc_agg(src2, dst2, z0, z1):
    mesh = plsc.VectorSubcoreMesh(core_axis_name="c", subcore_axis_name="s")

    @functools.partial(
        pl.kernel,
        out_type=[jax.ShapeDtypeStruct((16, TPN, 32), jnp.float32),
                  jax.ShapeDtypeStruct((16, TPN, 32), jnp.float32)],
        mesh=mesh,
        compiler_params=pltpu.CompilerParams(use_tc_tiling_on_sc=False),
        scratch_types=[
            pltpu.VMEM_SHARED((NPAD, 32), jnp.float32),
            pltpu.VMEM((8, 128), jnp.int32),
            pltpu.VMEM((8, 128), jnp.int32),
            pltpu.VMEM((128, 32), jnp.float32),
            pltpu.VMEM((ZROWS, 32), jnp.float32),
            pltpu.SemaphoreType.DMA,
        ],
    )
    def agg_kernel(src_hbm, dst_hbm, z0_hbm, z1_hbm, s0_hbm, s1_hbm,
                   acc, srcv, dstv, rows, zbuf, sem):
        c = lax.axis_index("c")
        sid = lax.axis_index("s")

        _fill_zero_rows(zbuf, ZROWS, 32)
        _zero_acc_slice(acc, zbuf, sid)
        plsc.subcore_barrier()

        def run(tab):
            def body(j, carry):
                row0 = sid * 400 + j * 8
                pltpu.sync_copy(src_hbm.at[pl.ds(row0, 8)], srcv)
                pltpu.sync_copy(dst_hbm.at[pl.ds(row0, 8)], dstv)
                for r in range(8):
                    pltpu.async_copy(tab.at[srcv.at[r]], rows, sem).wait()
                    pltpu.sync_copy(rows, acc.at[dstv.at[r]], add=True)
                return carry
            lax.fori_loop(0, 50, body, 0)

        @pl.when(c == 0)
        def _():
            run(z0_hbm)

        @pl.when(c == 1)
        def _():
            run(z1_hbm)

        plsc.subcore_barrier()
        base = sid * TPN

        @pl.when(c == 0)
        def _():
            pltpu.sync_copy(acc.at[pl.ds(base, TPN)], s0_hbm.at[sid])

        @pl.when(c == 1)
        def _():
            pltpu.sync_copy(acc.at[pl.ds(base, TPN)], s1_hbm.at[sid])

    s0, s1 = agg_kernel(src2, dst2, z0, z1)
    return s0.reshape(N, 32), s1.reshape(N, 32)


def _tc_prologue(xp, w1p, cnt):
    def body(x_ref, w_ref, cnt_ref, z0_ref, z1_ref, dis_ref):
        deg = jnp.sum(cnt_ref[...], axis=(0, 2)) + 1.0
        dis = lax.rsqrt(deg)[:, None]
        h = jnp.dot(x_ref[...], w_ref[...],
                    preferred_element_type=jnp.float32)
        z = dis * h
        z0_ref[...] = z[:, :32]
        z1_ref[...] = z[:, 32:]
        dis_ref[...] = jnp.broadcast_to(dis, (NB, 8))

    return pl.pallas_call(
        body,
        grid=(NGRID,),
        in_specs=[
            pl.BlockSpec((NB, 8), lambda i: (i, 0)),
            pl.BlockSpec((8, H), lambda i: (0, 0)),
            pl.BlockSpec((2, NB, 16), lambda i: (0, i, 0)),
        ],
        out_specs=[
            pl.BlockSpec((NB, 32), lambda i: (i, 0)),
            pl.BlockSpec((NB, 32), lambda i: (i, 0)),
            pl.BlockSpec((NB, 8), lambda i: (i, 0)),
        ],
        out_shape=[
            jax.ShapeDtypeStruct((N, 32), jnp.float32),
            jax.ShapeDtypeStruct((N, 32), jnp.float32),
            jax.ShapeDtypeStruct((N, 8), jnp.float32),
        ],
    )(xp, w1p, cnt)


def _tc_mid(s0, s1, z0, z1, dis8, br, w):
    def body(s0_ref, s1_ref, z0_ref, z1_ref, dis_ref, b_ref, w_ref,
             o0_ref, o1_ref):
        dis = dis_ref[...][:, :1]
        t = jnp.concatenate(
            [s0_ref[...] + z0_ref[...], s1_ref[...] + z1_ref[...]], axis=1)
        h = jnp.maximum(dis * t + b_ref[...], 0.0)
        zn = dis * jnp.dot(h, w_ref[...],
                           preferred_element_type=jnp.float32)
        o0_ref[...] = zn[:, :32]
        o1_ref[...] = zn[:, 32:]

    return pl.pallas_call(
        body,
        grid=(NGRID,),
        in_specs=[
            pl.BlockSpec((NB, 32), lambda i: (i, 0)),
            pl.BlockSpec((NB, 32), lambda i: (i, 0)),
            pl.BlockSpec((NB, 32), lambda i: (i, 0)),
            pl.BlockSpec((NB, 32), lambda i: (i, 0)),
            pl.BlockSpec((NB, 8), lambda i: (i, 0)),
            pl.BlockSpec((1, H), lambda i: (0, 0)),
            pl.BlockSpec((H, H), lambda i: (0, 0)),
        ],
        out_specs=[
            pl.BlockSpec((NB, 32), lambda i: (i, 0)),
            pl.BlockSpec((NB, 32), lambda i: (i, 0)),
        ],
        out_shape=[
            jax.ShapeDtypeStruct((N, 32), jnp.float32),
            jax.ShapeDtypeStruct((N, 32), jnp.float32),
        ],
    )(s0, s1, z0, z1, dis8, br, w)


def _tc_final(s0, s1, z0, z1, dis8, br, batch2, fcp):
    def body(s0_ref, s1_ref, z0_ref, z1_ref, dis_ref, b_ref, bat_ref,
             fc_ref, out_ref, acc_s, acc_c):
        i = pl.program_id(0)

        @pl.when(i == 0)
        def _():
            acc_s[...] = jnp.zeros((G, H), jnp.float32)
            acc_c[...] = jnp.zeros((1, G), jnp.float32)

        dis = dis_ref[...][:, :1]
        t = jnp.concatenate(
            [s0_ref[...] + z0_ref[...], s1_ref[...] + z1_ref[...]], axis=1)
        h = jnp.maximum(dis * t + b_ref[...], 0.0)
        gids = lax.broadcasted_iota(jnp.int32, (NB, G), 1)
        p = (bat_ref[...] == gids).astype(jnp.float32)
        acc_s[...] += lax.dot_general(
            p, h, (((0,), (0,)), ((), ())),
            preferred_element_type=jnp.float32)
        acc_c[...] += jnp.sum(p, axis=0, keepdims=True)

        @pl.when(i == NGRID - 1)
        def _():
            cnt = jnp.transpose(acc_c[...])
            pooled = acc_s[...] / jnp.maximum(cnt, 1.0)
            logits = jnp.dot(pooled, fc_ref[...],
                             preferred_element_type=jnp.float32)
            out_ref[...] = 1.0 / (1.0 + jnp.exp(-logits))

    return pl.pallas_call(
        body,
        grid=(NGRID,),
        in_specs=[
            pl.BlockSpec((NB, 32), lambda i: (i, 0)),
            pl.BlockSpec((NB, 32), lambda i: (i, 0)),
            pl.BlockSpec((NB, 32), lambda i: (i, 0)),
            pl.BlockSpec((NB, 32), lambda i: (i, 0)),
            pl.BlockSpec((NB, 8), lambda i: (i, 0)),
            pl.BlockSpec((1, H), lambda i: (0, 0)),
            pl.BlockSpec((NB, 1), lambda i: (i, 0)),
            pl.BlockSpec((H, 8), lambda i: (0, 0)),
        ],
        out_specs=pl.BlockSpec((G, 8), lambda i: (0, 0)),
        out_shape=jax.ShapeDtypeStruct((G, 8), jnp.float32),
        scratch_shapes=[
            pltpu.VMEM((G, H), jnp.float32),
            pltpu.VMEM((1, G), jnp.float32),
        ],
    )(s0, s1, z0, z1, dis8, br, batch2, fcp)


def kernel(x, edge_index, batch, W1, b1, W2, b2, W3, b3, fc_w):
    padn = EPAD - E
    src2 = jnp.concatenate(
        [edge_index[0], jnp.zeros((padn,), jnp.int32)]).reshape(EROWS, 128)
    dst2 = jnp.concatenate(
        [edge_index[1], jnp.full((padn,), N, jnp.int32)]).reshape(EROWS, 128)
    xp = jnp.pad(x, ((0, 0), (0, 4)))
    w1p = jnp.pad(W1, ((0, 4), (0, 0)))
    fcp = jnp.pad(fc_w, ((0, 0), (0, 7)))
    batch2 = batch.reshape(N, 1)

    cnt = _sc_deg(dst2)
    z0, z1, dis8 = _tc_prologue(xp, w1p, cnt)
    s0, s1 = _sc_agg(src2, dst2, z0, z1)
    z0, z1 = _tc_mid(s0, s1, z0, z1, dis8, b1.reshape(1, H), W2)
    s0, s1 = _sc_agg(src2, dst2, z0, z1)
    z0, z1 = _tc_mid(s0, s1, z0, z1, dis8, b2.reshape(1, H), W3)
    s0, s1 = _sc_agg(src2, dst2, z0, z1)
    out8 = _tc_final(s0, s1, z0, z1, dis8, b3.reshape(1, H), batch2, fcp)
    return out8[:, :1]


# speedup vs baseline: 14.5807x; 1.2714x over previous
import functools

import jax
import jax.numpy as jnp
from jax import lax
from jax.experimental import pallas as pl
from jax.experimental.pallas import tpu as pltpu
from jax.experimental.pallas import tpu_sc as plsc

N = 50000
E = 800000
H = 64
G = 64

EPAD = 819200
EROWS = EPAD // 64
NPAD = N + 8
NB = 1000
NGRID = N // NB
TPN = N // 16
ZROWS = TPN // 25


def _fill_zero_rows(ref, nrows, width):
    def body(j, carry):
        for w0 in range(0, width, 16):
            ref[j, pl.ds(w0, 16)] = jnp.zeros((16,), jnp.float32)
        return carry
    lax.fori_loop(0, nrows, body, 0)


def _zero_acc_slice(acc, zbuf, sid):
    base = sid * TPN

    def zcopy(k, carry):
        pltpu.sync_copy(zbuf, acc.at[pl.ds(base + k * ZROWS, ZROWS)])
        return carry
    lax.fori_loop(0, 25, zcopy, 0)


def _sc_deg(dst2):
    mesh = plsc.VectorSubcoreMesh(core_axis_name="c", subcore_axis_name="s")

    @functools.partial(
        pl.kernel,
        out_type=jax.ShapeDtypeStruct((2, 16, TPN, 16), jnp.float32),
        mesh=mesh,
        compiler_params=pltpu.CompilerParams(use_tc_tiling_on_sc=False),
        scratch_types=[
            pltpu.VMEM_SHARED((NPAD, 16), jnp.float32),
            pltpu.VMEM((8, 64), jnp.int32),
            pltpu.VMEM((64, 16), jnp.float32),
            pltpu.VMEM((ZROWS, 16), jnp.float32),
        ],
    )
    def deg_kernel(dst_hbm, cnt_hbm, acc, dstv, onesv, zbuf):
        c = lax.axis_index("c")
        sid = lax.axis_index("s")

        _fill_zero_rows(zbuf, ZROWS, 16)

        def fill_ones(j, carry):
            onesv[j, pl.ds(0, 16)] = jnp.ones((16,), jnp.float32)
            return carry
        lax.fori_loop(0, 64, fill_ones, 0)

        _zero_acc_slice(acc, zbuf, sid)
        plsc.subcore_barrier()

        def body(j, carry):
            row0 = c * (EROWS // 2) + sid * 400 + j * 8
            pltpu.sync_copy(dst_hbm.at[pl.ds(row0, 8)], dstv)
            for r in range(8):
                pltpu.sync_copy(onesv, acc.at[dstv.at[r]], add=True)
            return carry
        lax.fori_loop(0, 50, body, 0)

        plsc.subcore_barrier()
        pltpu.sync_copy(acc.at[pl.ds(sid * TPN, TPN)],
                        cnt_hbm.at[c, sid])

    return deg_kernel(dst2).reshape(2, N, 16)


def _sc_agg(src2, dst2, z0, z1):
    mesh = plsc.VectorSubcoreMesh(core_axis_name="c", subcore_axis_name="s")

    @functools.partial(
        pl.kernel,
        out_type=[jax.ShapeDtypeStruct((16, TPN, 32), jnp.float32),
                  jax.ShapeDtypeStruct((16, TPN, 32), jnp.float32)],
        mesh=mesh,
        compiler_params=pltpu.CompilerParams(use_tc_tiling_on_sc=False),
        scratch_types=[
            pltpu.VMEM_SHARED((NPAD, 32), jnp.float32),
            pltpu.VMEM((40, 64), jnp.int32),
            pltpu.VMEM((40, 64), jnp.int32),
            pltpu.VMEM((10, 64, 32), jnp.float32),
            pltpu.VMEM((ZROWS, 32), jnp.float32),
            pltpu.SemaphoreType.DMA,
            pltpu.SemaphoreType.DMA,
        ],
    )
    def agg_kernel(src_hbm, dst_hbm, z0_hbm, z1_hbm, s0_hbm, s1_hbm,
                   acc, srcv, dstv, rows, zbuf, sem_g, sem_s):
        c = lax.axis_index("c")
        sid = lax.axis_index("s")

        _fill_zero_rows(zbuf, ZROWS, 32)
        _zero_acc_slice(acc, zbuf, sid)
        plsc.subcore_barrier()

        def run(tab):
            def gather(row, b):
                return pltpu.async_copy(tab.at[srcv.at[row]],
                                        rows.at[b], sem_g)

            def scatter(row, b):
                return pltpu.async_copy(rows.at[b], acc.at[dstv.at[row]],
                                        sem_s, add=True)

            def body(j, carry):
                r0 = j * 10
                ga = [gather(r0 + k, k) for k in range(5)]
                for d in ga:
                    d.wait()
                sa = [scatter(r0 + k, k) for k in range(5)]
                gb = [gather(r0 + 5 + k, 5 + k) for k in range(5)]
                for d in sa:
                    d.wait()
                for d in gb:
                    d.wait()
                sb = [scatter(r0 + 5 + k, 5 + k) for k in range(5)]
                for d in sb:
                    d.wait()
                return carry

            def stage(st, carry):
                row0 = sid * 800 + st * 40
                pltpu.sync_copy(src_hbm.at[pl.ds(row0, 40)], srcv)
                pltpu.sync_copy(dst_hbm.at[pl.ds(row0, 40)], dstv)
                lax.fori_loop(0, 4, body, 0)
                return carry

            lax.fori_loop(0, 20, stage, 0)

        @pl.when(c == 0)
        def _():
            run(z0_hbm)

        @pl.when(c == 1)
        def _():
            run(z1_hbm)

        plsc.subcore_barrier()
        base = sid * TPN

        @pl.when(c == 0)
        def _():
            pltpu.sync_copy(acc.at[pl.ds(base, TPN)], s0_hbm.at[sid])

        @pl.when(c == 1)
        def _():
            pltpu.sync_copy(acc.at[pl.ds(base, TPN)], s1_hbm.at[sid])

    s0, s1 = agg_kernel(src2, dst2, z0, z1)
    return s0.reshape(N, 32), s1.reshape(N, 32)


def _tc_prologue(xp, w1p, cnt):
    def body(x_ref, w_ref, cnt_ref, z0_ref, z1_ref, dis_ref):
        deg = jnp.sum(cnt_ref[...], axis=(0, 2)) + 1.0
        dis = lax.rsqrt(deg)[:, None]
        h = jnp.dot(x_ref[...], w_ref[...],
                    preferred_element_type=jnp.float32)
        z = dis * h
        z0_ref[...] = z[:, :32]
        z1_ref[...] = z[:, 32:]
        dis_ref[...] = jnp.broadcast_to(dis, (NB, 8))

    return pl.pallas_call(
        body,
        grid=(NGRID,),
        in_specs=[
            pl.BlockSpec((NB, 8), lambda i: (i, 0)),
            pl.BlockSpec((8, H), lambda i: (0, 0)),
            pl.BlockSpec((2, NB, 16), lambda i: (0, i, 0)),
        ],
        out_specs=[
            pl.BlockSpec((NB, 32), lambda i: (i, 0)),
            pl.BlockSpec((NB, 32), lambda i: (i, 0)),
            pl.BlockSpec((NB, 8), lambda i: (i, 0)),
        ],
        out_shape=[
            jax.ShapeDtypeStruct((N, 32), jnp.float32),
            jax.ShapeDtypeStruct((N, 32), jnp.float32),
            jax.ShapeDtypeStruct((N, 8), jnp.float32),
        ],
    )(xp, w1p, cnt)


def _tc_mid(s0, s1, z0, z1, dis8, br, w):
    def body(s0_ref, s1_ref, z0_ref, z1_ref, dis_ref, b_ref, w_ref,
             o0_ref, o1_ref):
        dis = dis_ref[...][:, :1]
        t = jnp.concatenate(
            [s0_ref[...] + z0_ref[...], s1_ref[...] + z1_ref[...]], axis=1)
        h = jnp.maximum(dis * t + b_ref[...], 0.0)
        zn = dis * jnp.dot(h, w_ref[...],
                           preferred_element_type=jnp.float32)
        o0_ref[...] = zn[:, :32]
        o1_ref[...] = zn[:, 32:]

    return pl.pallas_call(
        body,
        grid=(NGRID,),
        in_specs=[
            pl.BlockSpec((NB, 32), lambda i: (i, 0)),
            pl.BlockSpec((NB, 32), lambda i: (i, 0)),
            pl.BlockSpec((NB, 32), lambda i: (i, 0)),
            pl.BlockSpec((NB, 32), lambda i: (i, 0)),
            pl.BlockSpec((NB, 8), lambda i: (i, 0)),
            pl.BlockSpec((1, H), lambda i: (0, 0)),
            pl.BlockSpec((H, H), lambda i: (0, 0)),
        ],
        out_specs=[
            pl.BlockSpec((NB, 32), lambda i: (i, 0)),
            pl.BlockSpec((NB, 32), lambda i: (i, 0)),
        ],
        out_shape=[
            jax.ShapeDtypeStruct((N, 32), jnp.float32),
            jax.ShapeDtypeStruct((N, 32), jnp.float32),
        ],
    )(s0, s1, z0, z1, dis8, br, w)


def _tc_final(s0, s1, z0, z1, dis8, br, batch2, fcp):
    def body(s0_ref, s1_ref, z0_ref, z1_ref, dis_ref, b_ref, bat_ref,
             fc_ref, out_ref, acc_s, acc_c):
        i = pl.program_id(0)

        @pl.when(i == 0)
        def _():
            acc_s[...] = jnp.zeros((G, H), jnp.float32)
            acc_c[...] = jnp.zeros((1, G), jnp.float32)

        dis = dis_ref[...][:, :1]
        t = jnp.concatenate(
            [s0_ref[...] + z0_ref[...], s1_ref[...] + z1_ref[...]], axis=1)
        h = jnp.maximum(dis * t + b_ref[...], 0.0)
        gids = lax.broadcasted_iota(jnp.int32, (NB, G), 1)
        p = (bat_ref[...] == gids).astype(jnp.float32)
        acc_s[...] += lax.dot_general(
            p, h, (((0,), (0,)), ((), ())),
            preferred_element_type=jnp.float32)
        acc_c[...] += jnp.sum(p, axis=0, keepdims=True)

        @pl.when(i == NGRID - 1)
        def _():
            cnt = jnp.transpose(acc_c[...])
            pooled = acc_s[...] / jnp.maximum(cnt, 1.0)
            logits = jnp.dot(pooled, fc_ref[...],
                             preferred_element_type=jnp.float32)
            out_ref[...] = 1.0 / (1.0 + jnp.exp(-logits))

    return pl.pallas_call(
        body,
        grid=(NGRID,),
        in_specs=[
            pl.BlockSpec((NB, 32), lambda i: (i, 0)),
            pl.BlockSpec((NB, 32), lambda i: (i, 0)),
            pl.BlockSpec((NB, 32), lambda i: (i, 0)),
            pl.BlockSpec((NB, 32), lambda i: (i, 0)),
            pl.BlockSpec((NB, 8), lambda i: (i, 0)),
            pl.BlockSpec((1, H), lambda i: (0, 0)),
            pl.BlockSpec((NB, 1), lambda i: (i, 0)),
            pl.BlockSpec((H, 8), lambda i: (0, 0)),
        ],
        out_specs=pl.BlockSpec((G, 8), lambda i: (0, 0)),
        out_shape=jax.ShapeDtypeStruct((G, 8), jnp.float32),
        scratch_shapes=[
            pltpu.VMEM((G, H), jnp.float32),
            pltpu.VMEM((1, G), jnp.float32),
        ],
    )(s0, s1, z0, z1, dis8, br, batch2, fcp)


def kernel(x, edge_index, batch, W1, b1, W2, b2, W3, b3, fc_w):
    padn = EPAD - E
    src2 = jnp.concatenate(
        [edge_index[0], jnp.zeros((padn,), jnp.int32)]).reshape(EROWS, 64)
    dst2 = jnp.concatenate(
        [edge_index[1], jnp.full((padn,), N, jnp.int32)]).reshape(EROWS, 64)
    xp = jnp.pad(x, ((0, 0), (0, 4)))
    w1p = jnp.pad(W1, ((0, 4), (0, 0)))
    fcp = jnp.pad(fc_w, ((0, 0), (0, 7)))
    batch2 = batch.reshape(N, 1)

    cnt = _sc_deg(dst2)
    z0, z1, dis8 = _tc_prologue(xp, w1p, cnt)
    s0, s1 = _sc_agg(src2, dst2, z0, z1)
    z0, z1 = _tc_mid(s0, s1, z0, z1, dis8, b1.reshape(1, H), W2)
    s0, s1 = _sc_agg(src2, dst2, z0, z1)
    z0, z1 = _tc_mid(s0, s1, z0, z1, dis8, b2.reshape(1, H), W3)
    s0, s1 = _sc_agg(src2, dst2, z0, z1)
    out8 = _tc_final(s0, s1, z0, z1, dis8, b3.reshape(1, H), batch2, fcp)
    return out8[:, :1]


# speedup vs baseline: 15.7350x; 1.0792x over previous
import functools

import jax
import jax.numpy as jnp
from jax import lax
from jax.experimental import pallas as pl
from jax.experimental.pallas import tpu as pltpu
from jax.experimental.pallas import tpu_sc as plsc

N = 50000
E = 800000
H = 64
G = 64

EPAD = 819200
EROWS = EPAD // 64
NPAD = N + 8
NB = 1000
NGRID = N // NB
TPN = N // 16
ZROWS = TPN // 25


def _fill_zero_rows(ref, nrows, width):
    def body(j, carry):
        for w0 in range(0, width, 16):
            ref[j, pl.ds(w0, 16)] = jnp.zeros((16,), jnp.float32)
        return carry
    lax.fori_loop(0, nrows, body, 0)


def _zero_acc_slice(acc, zbuf, sid):
    base = sid * TPN

    def zcopy(k, carry):
        pltpu.sync_copy(zbuf, acc.at[pl.ds(base + k * ZROWS, ZROWS)])
        return carry
    lax.fori_loop(0, 25, zcopy, 0)


def _sc_deg(dst2):
    mesh = plsc.VectorSubcoreMesh(core_axis_name="c", subcore_axis_name="s")

    @functools.partial(
        pl.kernel,
        out_type=jax.ShapeDtypeStruct((2, 16, TPN, 16), jnp.float32),
        mesh=mesh,
        compiler_params=pltpu.CompilerParams(use_tc_tiling_on_sc=False),
        scratch_types=[
            pltpu.VMEM_SHARED((NPAD, 16), jnp.float32),
            pltpu.VMEM((8, 64), jnp.int32),
            pltpu.VMEM((64, 16), jnp.float32),
            pltpu.VMEM((ZROWS, 16), jnp.float32),
        ],
    )
    def deg_kernel(dst_hbm, cnt_hbm, acc, dstv, onesv, zbuf):
        c = lax.axis_index("c")
        sid = lax.axis_index("s")

        _fill_zero_rows(zbuf, ZROWS, 16)

        def fill_ones(j, carry):
            onesv[j, pl.ds(0, 16)] = jnp.ones((16,), jnp.float32)
            return carry
        lax.fori_loop(0, 64, fill_ones, 0)

        _zero_acc_slice(acc, zbuf, sid)
        plsc.subcore_barrier()

        def body(j, carry):
            row0 = c * (EROWS // 2) + sid * 400 + j * 8
            pltpu.sync_copy(dst_hbm.at[pl.ds(row0, 8)], dstv)
            for r in range(8):
                pltpu.sync_copy(onesv, acc.at[dstv.at[r]], add=True)
            return carry
        lax.fori_loop(0, 50, body, 0)

        plsc.subcore_barrier()
        pltpu.sync_copy(acc.at[pl.ds(sid * TPN, TPN)],
                        cnt_hbm.at[c, sid])

    return deg_kernel(dst2).reshape(2, N, 16)


def _sc_agg(src2, dst2, z0, z1):
    mesh = plsc.VectorSubcoreMesh(core_axis_name="c", subcore_axis_name="s")

    @functools.partial(
        pl.kernel,
        out_type=[jax.ShapeDtypeStruct((16, TPN, 32), jnp.float32),
                  jax.ShapeDtypeStruct((16, TPN, 32), jnp.float32)],
        mesh=mesh,
        compiler_params=pltpu.CompilerParams(use_tc_tiling_on_sc=False),
        scratch_types=[
            pltpu.VMEM_SHARED((NPAD, 32), jnp.float32),
            pltpu.VMEM((40, 64), jnp.int32),
            pltpu.VMEM((40, 64), jnp.int32),
            pltpu.VMEM((10, 64, 32), jnp.float32),
            pltpu.VMEM((ZROWS, 32), jnp.float32),
            pltpu.SemaphoreType.DMA,
            pltpu.SemaphoreType.DMA,
        ],
    )
    def agg_kernel(src_hbm, dst_hbm, z0_hbm, z1_hbm, s0_hbm, s1_hbm,
                   acc, srcv, dstv, rows, zbuf, sem_g, sem_s):
        c = lax.axis_index("c")
        sid = lax.axis_index("s")

        _fill_zero_rows(zbuf, ZROWS, 32)
        _zero_acc_slice(acc, zbuf, sid)
        plsc.subcore_barrier()

        def run(tab):
            def gather(row, b):
                return pltpu.async_copy(tab.at[srcv.at[row]],
                                        rows.at[b], sem_g)

            def scatter(row, b):
                return pltpu.async_copy(rows.at[b], acc.at[dstv.at[row]],
                                        sem_s, add=True)

            def body(j, carry):
                r0 = j * 20

                def G(t):
                    b0 = (t % 2) * 5
                    return [gather(r0 + t * 5 + k, b0 + k)
                            for k in range(5)]

                def S(t):
                    b0 = (t % 2) * 5
                    return [scatter(r0 + t * 5 + k, b0 + k)
                            for k in range(5)]

                def drain(ds):
                    for d in ds:
                        d.wait()

                g0 = G(0)
                g1 = G(1)
                drain(g0)
                s0 = S(0)
                drain(g1)
                s1 = S(1)
                drain(s0)
                g2 = G(2)
                drain(s1)
                g3 = G(3)
                drain(g2)
                s2 = S(2)
                drain(g3)
                s3 = S(3)
                drain(s2)
                drain(s3)
                return carry

            def stage(st, carry):
                row0 = sid * 800 + st * 40
                pltpu.sync_copy(src_hbm.at[pl.ds(row0, 40)], srcv)
                pltpu.sync_copy(dst_hbm.at[pl.ds(row0, 40)], dstv)
                lax.fori_loop(0, 2, body, 0)
                return carry

            lax.fori_loop(0, 20, stage, 0)

        @pl.when(c == 0)
        def _():
            run(z0_hbm)

        @pl.when(c == 1)
        def _():
            run(z1_hbm)

        plsc.subcore_barrier()
        base = sid * TPN

        @pl.when(c == 0)
        def _():
            pltpu.sync_copy(acc.at[pl.ds(base, TPN)], s0_hbm.at[sid])

        @pl.when(c == 1)
        def _():
            pltpu.sync_copy(acc.at[pl.ds(base, TPN)], s1_hbm.at[sid])

    s0, s1 = agg_kernel(src2, dst2, z0, z1)
    return s0.reshape(N, 32), s1.reshape(N, 32)


def _tc_prologue(xp, w1p, cnt):
    def body(x_ref, w_ref, cnt_ref, z0_ref, z1_ref, dis_ref):
        deg = jnp.sum(cnt_ref[...], axis=(0, 2)) + 1.0
        dis = lax.rsqrt(deg)[:, None]
        h = jnp.dot(x_ref[...], w_ref[...],
                    preferred_element_type=jnp.float32)
        z = dis * h
        z0_ref[...] = z[:, :32]
        z1_ref[...] = z[:, 32:]
        dis_ref[...] = jnp.broadcast_to(dis, (NB, 8))

    return pl.pallas_call(
        body,
        grid=(NGRID,),
        in_specs=[
            pl.BlockSpec((NB, 8), lambda i: (i, 0)),
            pl.BlockSpec((8, H), lambda i: (0, 0)),
            pl.BlockSpec((2, NB, 16), lambda i: (0, i, 0)),
        ],
        out_specs=[
            pl.BlockSpec((NB, 32), lambda i: (i, 0)),
            pl.BlockSpec((NB, 32), lambda i: (i, 0)),
            pl.BlockSpec((NB, 8), lambda i: (i, 0)),
        ],
        out_shape=[
            jax.ShapeDtypeStruct((N, 32), jnp.float32),
            jax.ShapeDtypeStruct((N, 32), jnp.float32),
            jax.ShapeDtypeStruct((N, 8), jnp.float32),
        ],
    )(xp, w1p, cnt)


def _tc_mid(s0, s1, z0, z1, dis8, br, w):
    def body(s0_ref, s1_ref, z0_ref, z1_ref, dis_ref, b_ref, w_ref,
             o0_ref, o1_ref):
        dis = dis_ref[...][:, :1]
        t = jnp.concatenate(
            [s0_ref[...] + z0_ref[...], s1_ref[...] + z1_ref[...]], axis=1)
        h = jnp.maximum(dis * t + b_ref[...], 0.0)
        zn = dis * jnp.dot(h, w_ref[...],
                           preferred_element_type=jnp.float32)
        o0_ref[...] = zn[:, :32]
        o1_ref[...] = zn[:, 32:]

    return pl.pallas_call(
        body,
        grid=(NGRID,),
        in_specs=[
            pl.BlockSpec((NB, 32), lambda i: (i, 0)),
            pl.BlockSpec((NB, 32), lambda i: (i, 0)),
            pl.BlockSpec((NB, 32), lambda i: (i, 0)),
            pl.BlockSpec((NB, 32), lambda i: (i, 0)),
            pl.BlockSpec((NB, 8), lambda i: (i, 0)),
            pl.BlockSpec((1, H), lambda i: (0, 0)),
            pl.BlockSpec((H, H), lambda i: (0, 0)),
        ],
        out_specs=[
            pl.BlockSpec((NB, 32), lambda i: (i, 0)),
            pl.BlockSpec((NB, 32), lambda i: (i, 0)),
        ],
        out_shape=[
            jax.ShapeDtypeStruct((N, 32), jnp.float32),
            jax.ShapeDtypeStruct((N, 32), jnp.float32),
        ],
    )(s0, s1, z0, z1, dis8, br, w)


def _tc_final(s0, s1, z0, z1, dis8, br, batch2, fcp):
    def body(s0_ref, s1_ref, z0_ref, z1_ref, dis_ref, b_ref, bat_ref,
             fc_ref, out_ref, acc_s, acc_c):
        i = pl.program_id(0)

        @pl.when(i == 0)
        def _():
            acc_s[...] = jnp.zeros((G, H), jnp.float32)
            acc_c[...] = jnp.zeros((1, G), jnp.float32)

        dis = dis_ref[...][:, :1]
        t = jnp.concatenate(
            [s0_ref[...] + z0_ref[...], s1_ref[...] + z1_ref[...]], axis=1)
        h = jnp.maximum(dis * t + b_ref[...], 0.0)
        gids = lax.broadcasted_iota(jnp.int32, (NB, G), 1)
        p = (bat_ref[...] == gids).astype(jnp.float32)
        acc_s[...] += lax.dot_general(
            p, h, (((0,), (0,)), ((), ())),
            preferred_element_type=jnp.float32)
        acc_c[...] += jnp.sum(p, axis=0, keepdims=True)

        @pl.when(i == NGRID - 1)
        def _():
            cnt = jnp.transpose(acc_c[...])
            pooled = acc_s[...] / jnp.maximum(cnt, 1.0)
            logits = jnp.dot(pooled, fc_ref[...],
                             preferred_element_type=jnp.float32)
            out_ref[...] = 1.0 / (1.0 + jnp.exp(-logits))

    return pl.pallas_call(
        body,
        grid=(NGRID,),
        in_specs=[
            pl.BlockSpec((NB, 32), lambda i: (i, 0)),
            pl.BlockSpec((NB, 32), lambda i: (i, 0)),
            pl.BlockSpec((NB, 32), lambda i: (i, 0)),
            pl.BlockSpec((NB, 32), lambda i: (i, 0)),
            pl.BlockSpec((NB, 8), lambda i: (i, 0)),
            pl.BlockSpec((1, H), lambda i: (0, 0)),
            pl.BlockSpec((NB, 1), lambda i: (i, 0)),
            pl.BlockSpec((H, 8), lambda i: (0, 0)),
        ],
        out_specs=pl.BlockSpec((G, 8), lambda i: (0, 0)),
        out_shape=jax.ShapeDtypeStruct((G, 8), jnp.float32),
        scratch_shapes=[
            pltpu.VMEM((G, H), jnp.float32),
            pltpu.VMEM((1, G), jnp.float32),
        ],
    )(s0, s1, z0, z1, dis8, br, batch2, fcp)


def kernel(x, edge_index, batch, W1, b1, W2, b2, W3, b3, fc_w):
    padn = EPAD - E
    src2 = jnp.concatenate(
        [edge_index[0], jnp.zeros((padn,), jnp.int32)]).reshape(EROWS, 64)
    dst2 = jnp.concatenate(
        [edge_index[1], jnp.full((padn,), N, jnp.int32)]).reshape(EROWS, 64)
    xp = jnp.pad(x, ((0, 0), (0, 4)))
    w1p = jnp.pad(W1, ((0, 4), (0, 0)))
    fcp = jnp.pad(fc_w, ((0, 0), (0, 7)))
    batch2 = batch.reshape(N, 1)

    cnt = _sc_deg(dst2)
    z0, z1, dis8 = _tc_prologue(xp, w1p, cnt)
    s0, s1 = _sc_agg(src2, dst2, z0, z1)
    z0, z1 = _tc_mid(s0, s1, z0, z1, dis8, b1.reshape(1, H), W2)
    s0, s1 = _sc_agg(src2, dst2, z0, z1)
    z0, z1 = _tc_mid(s0, s1, z0, z1, dis8, b2.reshape(1, H), W3)
    s0, s1 = _sc_agg(src2, dst2, z0, z1)
    out8 = _tc_final(s0, s1, z0, z1, dis8, b3.reshape(1, H), batch2, fcp)
    return out8[:, :1]


# speedup vs baseline: 16.2162x; 1.0306x over previous
import functools

import jax
import jax.numpy as jnp
from jax import lax
from jax.experimental import pallas as pl
from jax.experimental.pallas import tpu as pltpu
from jax.experimental.pallas import tpu_sc as plsc

N = 50000
E = 800000
H = 64
G = 64

EPAD = 819200
EROWS = EPAD // 64
NPAD = N + 8
NB = 1000
NGRID = N // NB
TPN = N // 16
ZROWS = TPN // 25


def _fill_zero_rows(ref, nrows, width):
    def body(j, carry):
        for w0 in range(0, width, 16):
            ref[j, pl.ds(w0, 16)] = jnp.zeros((16,), jnp.float32)
        return carry
    lax.fori_loop(0, nrows, body, 0)


def _zero_acc_slice(acc, zbuf, sid, sem):
    base = sid * TPN

    def zcopy(k, carry):
        pltpu.async_copy(zbuf, acc.at[pl.ds(base + k * ZROWS, ZROWS)], sem)
        return carry
    lax.fori_loop(0, 25, zcopy, 0)

    def zdrain(k, carry):
        pltpu.make_async_copy(zbuf, acc.at[pl.ds(base, ZROWS)], sem).wait()
        return carry
    lax.fori_loop(0, 25, zdrain, 0)


def _sc_deg(dst2):
    mesh = plsc.VectorSubcoreMesh(core_axis_name="c", subcore_axis_name="s")

    @functools.partial(
        pl.kernel,
        out_type=jax.ShapeDtypeStruct((2, 16, TPN, 16), jnp.float32),
        mesh=mesh,
        compiler_params=pltpu.CompilerParams(use_tc_tiling_on_sc=False),
        scratch_types=[
            pltpu.VMEM_SHARED((NPAD, 16), jnp.float32),
            pltpu.VMEM((8, 64), jnp.int32),
            pltpu.VMEM((64, 16), jnp.float32),
            pltpu.VMEM((ZROWS, 16), jnp.float32),
            pltpu.SemaphoreType.DMA,
        ],
    )
    def deg_kernel(dst_hbm, cnt_hbm, acc, dstv, onesv, zbuf, sem):
        c = lax.axis_index("c")
        sid = lax.axis_index("s")

        _fill_zero_rows(zbuf, ZROWS, 16)

        def fill_ones(j, carry):
            onesv[j, pl.ds(0, 16)] = jnp.ones((16,), jnp.float32)
            return carry
        lax.fori_loop(0, 64, fill_ones, 0)

        _zero_acc_slice(acc, zbuf, sid, sem)
        plsc.subcore_barrier()

        def body(j, carry):
            row0 = c * (EROWS // 2) + sid * 400 + j * 8
            pltpu.sync_copy(dst_hbm.at[pl.ds(row0, 8)], dstv)
            ds = [pltpu.async_copy(onesv, acc.at[dstv.at[r]], sem,
                                   add=True)
                  for r in range(8)]
            for d in ds:
                d.wait()
            return carry
        lax.fori_loop(0, 50, body, 0)

        plsc.subcore_barrier()
        pltpu.sync_copy(acc.at[pl.ds(sid * TPN, TPN)],
                        cnt_hbm.at[c, sid])

    return deg_kernel(dst2).reshape(2, N, 16)


def _sc_agg(src2, dst2, z0, z1):
    mesh = plsc.VectorSubcoreMesh(core_axis_name="c", subcore_axis_name="s")

    @functools.partial(
        pl.kernel,
        out_type=[jax.ShapeDtypeStruct((16, TPN, 32), jnp.float32),
                  jax.ShapeDtypeStruct((16, TPN, 32), jnp.float32)],
        mesh=mesh,
        compiler_params=pltpu.CompilerParams(use_tc_tiling_on_sc=False),
        scratch_types=[
            pltpu.VMEM_SHARED((NPAD, 32), jnp.float32),
            pltpu.VMEM((40, 64), jnp.int32),
            pltpu.VMEM((40, 64), jnp.int32),
            pltpu.VMEM((10, 64, 32), jnp.float32),
            pltpu.VMEM((ZROWS, 32), jnp.float32),
            pltpu.SemaphoreType.DMA,
            pltpu.SemaphoreType.DMA,
        ],
    )
    def agg_kernel(src_hbm, dst_hbm, z0_hbm, z1_hbm, s0_hbm, s1_hbm,
                   acc, srcv, dstv, rows, zbuf, sem_g, sem_s):
        c = lax.axis_index("c")
        sid = lax.axis_index("s")

        _fill_zero_rows(zbuf, ZROWS, 32)
        _zero_acc_slice(acc, zbuf, sid, sem_s)
        plsc.subcore_barrier()

        def run(tab):
            def gather(row, b):
                return pltpu.async_copy(tab.at[srcv.at[row]],
                                        rows.at[b], sem_g)

            def scatter(row, b):
                return pltpu.async_copy(rows.at[b], acc.at[dstv.at[row]],
                                        sem_s, add=True)

            def body(j, carry):
                r0 = j * 20

                def G(t):
                    b0 = (t % 2) * 5
                    return [gather(r0 + t * 5 + k, b0 + k)
                            for k in range(5)]

                def S(t):
                    b0 = (t % 2) * 5
                    return [scatter(r0 + t * 5 + k, b0 + k)
                            for k in range(5)]

                def drain(ds):
                    for d in ds:
                        d.wait()

                g0 = G(0)
                g1 = G(1)
                drain(g0)
                s0 = S(0)
                drain(g1)
                s1 = S(1)
                drain(s0)
                g2 = G(2)
                drain(s1)
                g3 = G(3)
                drain(g2)
                s2 = S(2)
                drain(g3)
                s3 = S(3)
                drain(s2)
                drain(s3)
                return carry

            def stage(st, carry):
                row0 = sid * 800 + st * 40
                pltpu.sync_copy(src_hbm.at[pl.ds(row0, 40)], srcv)
                pltpu.sync_copy(dst_hbm.at[pl.ds(row0, 40)], dstv)
                lax.fori_loop(0, 2, body, 0)
                return carry

            lax.fori_loop(0, 20, stage, 0)

        @pl.when(c == 0)
        def _():
            run(z0_hbm)

        @pl.when(c == 1)
        def _():
            run(z1_hbm)

        plsc.subcore_barrier()
        base = sid * TPN

        @pl.when(c == 0)
        def _():
            pltpu.sync_copy(acc.at[pl.ds(base, TPN)], s0_hbm.at[sid])

        @pl.when(c == 1)
        def _():
            pltpu.sync_copy(acc.at[pl.ds(base, TPN)], s1_hbm.at[sid])

    s0, s1 = agg_kernel(src2, dst2, z0, z1)
    return s0.reshape(N, 32), s1.reshape(N, 32)


def _tc_pre(xp, w1p):
    def body(x_ref, w_ref, y_ref):
        y_ref[...] = jnp.dot(x_ref[...], w_ref[...],
                             preferred_element_type=jnp.float32)

    return pl.pallas_call(
        body,
        grid=(NGRID,),
        in_specs=[
            pl.BlockSpec((NB, 8), lambda i: (i, 0)),
            pl.BlockSpec((8, H), lambda i: (0, 0)),
        ],
        out_specs=pl.BlockSpec((NB, H), lambda i: (i, 0)),
        out_shape=jax.ShapeDtypeStruct((N, H), jnp.float32),
    )(xp, w1p)


def _tc_scale(y1, cnt):
    def body(y_ref, cnt_ref, z0_ref, z1_ref, dis_ref):
        deg = jnp.sum(cnt_ref[...], axis=(0, 2)) + 1.0
        dis = lax.rsqrt(deg)[:, None]
        z = dis * y_ref[...]
        z0_ref[...] = z[:, :32]
        z1_ref[...] = z[:, 32:]
        dis_ref[...] = jnp.broadcast_to(dis, (NB, 8))

    return pl.pallas_call(
        body,
        grid=(NGRID,),
        in_specs=[
            pl.BlockSpec((NB, H), lambda i: (i, 0)),
            pl.BlockSpec((2, NB, 16), lambda i: (0, i, 0)),
        ],
        out_specs=[
            pl.BlockSpec((NB, 32), lambda i: (i, 0)),
            pl.BlockSpec((NB, 32), lambda i: (i, 0)),
            pl.BlockSpec((NB, 8), lambda i: (i, 0)),
        ],
        out_shape=[
            jax.ShapeDtypeStruct((N, 32), jnp.float32),
            jax.ShapeDtypeStruct((N, 32), jnp.float32),
            jax.ShapeDtypeStruct((N, 8), jnp.float32),
        ],
    )(y1, cnt)


def _tc_mid(s0, s1, z0, z1, dis8, br, w):
    def body(s0_ref, s1_ref, z0_ref, z1_ref, dis_ref, b_ref, w_ref,
             o0_ref, o1_ref):
        dis = dis_ref[...][:, :1]
        t = jnp.concatenate(
            [s0_ref[...] + z0_ref[...], s1_ref[...] + z1_ref[...]], axis=1)
        h = jnp.maximum(dis * t + b_ref[...], 0.0)
        zn = dis * jnp.dot(h, w_ref[...],
                           preferred_element_type=jnp.float32)
        o0_ref[...] = zn[:, :32]
        o1_ref[...] = zn[:, 32:]

    return pl.pallas_call(
        body,
        grid=(NGRID,),
        in_specs=[
            pl.BlockSpec((NB, 32), lambda i: (i, 0)),
            pl.BlockSpec((NB, 32), lambda i: (i, 0)),
            pl.BlockSpec((NB, 32), lambda i: (i, 0)),
            pl.BlockSpec((NB, 32), lambda i: (i, 0)),
            pl.BlockSpec((NB, 8), lambda i: (i, 0)),
            pl.BlockSpec((1, H), lambda i: (0, 0)),
            pl.BlockSpec((H, H), lambda i: (0, 0)),
        ],
        out_specs=[
            pl.BlockSpec((NB, 32), lambda i: (i, 0)),
            pl.BlockSpec((NB, 32), lambda i: (i, 0)),
        ],
        out_shape=[
            jax.ShapeDtypeStruct((N, 32), jnp.float32),
            jax.ShapeDtypeStruct((N, 32), jnp.float32),
        ],
    )(s0, s1, z0, z1, dis8, br, w)


def _tc_final(s0, s1, z0, z1, dis8, br, batch2, fcp):
    def body(s0_ref, s1_ref, z0_ref, z1_ref, dis_ref, b_ref, bat_ref,
             fc_ref, out_ref, acc_s, acc_c):
        i = pl.program_id(0)

        @pl.when(i == 0)
        def _():
            acc_s[...] = jnp.zeros((G, H), jnp.float32)
            acc_c[...] = jnp.zeros((1, G), jnp.float32)

        dis = dis_ref[...][:, :1]
        t = jnp.concatenate(
            [s0_ref[...] + z0_ref[...], s1_ref[...] + z1_ref[...]], axis=1)
        h = jnp.maximum(dis * t + b_ref[...], 0.0)
        gids = lax.broadcasted_iota(jnp.int32, (NB, G), 1)
        p = (bat_ref[...] == gids).astype(jnp.float32)
        acc_s[...] += lax.dot_general(
            p, h, (((0,), (0,)), ((), ())),
            preferred_element_type=jnp.float32)
        acc_c[...] += jnp.sum(p, axis=0, keepdims=True)

        @pl.when(i == NGRID - 1)
        def _():
            cnt = jnp.transpose(acc_c[...])
            pooled = acc_s[...] / jnp.maximum(cnt, 1.0)
            logits = jnp.dot(pooled, fc_ref[...],
                             preferred_element_type=jnp.float32)
            out_ref[...] = 1.0 / (1.0 + jnp.exp(-logits))

    return pl.pallas_call(
        body,
        grid=(NGRID,),
        in_specs=[
            pl.BlockSpec((NB, 32), lambda i: (i, 0)),
            pl.BlockSpec((NB, 32), lambda i: (i, 0)),
            pl.BlockSpec((NB, 32), lambda i: (i, 0)),
            pl.BlockSpec((NB, 32), lambda i: (i, 0)),
            pl.BlockSpec((NB, 8), lambda i: (i, 0)),
            pl.BlockSpec((1, H), lambda i: (0, 0)),
            pl.BlockSpec((NB, 1), lambda i: (i, 0)),
            pl.BlockSpec((H, 8), lambda i: (0, 0)),
        ],
        out_specs=pl.BlockSpec((G, 8), lambda i: (0, 0)),
        out_shape=jax.ShapeDtypeStruct((G, 8), jnp.float32),
        scratch_shapes=[
            pltpu.VMEM((G, H), jnp.float32),
            pltpu.VMEM((1, G), jnp.float32),
        ],
    )(s0, s1, z0, z1, dis8, br, batch2, fcp)


def kernel(x, edge_index, batch, W1, b1, W2, b2, W3, b3, fc_w):
    padn = EPAD - E
    src2 = jnp.concatenate(
        [edge_index[0], jnp.zeros((padn,), jnp.int32)]).reshape(EROWS, 64)
    dst2 = jnp.concatenate(
        [edge_index[1], jnp.full((padn,), N, jnp.int32)]).reshape(EROWS, 64)
    xp = jnp.pad(x, ((0, 0), (0, 4)))
    w1p = jnp.pad(W1, ((0, 4), (0, 0)))
    fcp = jnp.pad(fc_w, ((0, 0), (0, 7)))
    batch2 = batch.reshape(N, 1)

    cnt = _sc_deg(dst2)
    y1 = _tc_pre(xp, w1p)
    z0, z1, dis8 = _tc_scale(y1, cnt)
    s0, s1 = _sc_agg(src2, dst2, z0, z1)
    z0, z1 = _tc_mid(s0, s1, z0, z1, dis8, b1.reshape(1, H), W2)
    s0, s1 = _sc_agg(src2, dst2, z0, z1)
    z0, z1 = _tc_mid(s0, s1, z0, z1, dis8, b2.reshape(1, H), W3)
    s0, s1 = _sc_agg(src2, dst2, z0, z1)
    out8 = _tc_final(s0, s1, z0, z1, dis8, b3.reshape(1, H), batch2, fcp)
    return out8[:, :1]
